# baseline (speedup 1.0000x reference)
import functools

import jax
import jax.numpy as jnp
from jax import lax
from jax.experimental import pallas as pl
from jax.experimental.pallas import tpu as pltpu
from jax.experimental.pallas import tpu_sc as plsc

_CHUNK = 128
_NW = 32
_LANES = 16


def _sig(x):
    return 1.0 / (1.0 + jnp.exp(-x))


_DN = (((1,), (1,)), ((), ()))


def _proj_kernel(nf_ref, w1s_ref, w1d_ref, ps_ref, pd_ref):
    nfb = nf_ref[...]
    ps_ref[...] = lax.dot_general(nfb, w1s_ref[...], _DN,
                                  preferred_element_type=jnp.float32)
    pd_ref[...] = lax.dot_general(nfb, w1d_ref[...], _DN,
                                  preferred_element_type=jnp.float32)


def _proj(nf, w1s, w1d):
    n, d = nf.shape
    nb = 2000 if n % 2000 == 0 else n
    return pl.pallas_call(
        _proj_kernel,
        grid=(n // nb,),
        in_specs=[pl.BlockSpec((nb, d), lambda i: (i, 0)),
                  pl.BlockSpec((d, d), lambda i: (0, 0)),
                  pl.BlockSpec((d, d), lambda i: (0, 0))],
        out_specs=[pl.BlockSpec((nb, d), lambda i: (i, 0)),
                   pl.BlockSpec((nb, d), lambda i: (i, 0))],
        out_shape=[jax.ShapeDtypeStruct((n, d), jnp.float32)] * 2,
    )(nf, w1s, w1d)


def _edge_kernel(h0_ref, ef_ref, w1e_ref, b1_ref, lng_ref, lnb_ref,
                 gw_ref, gb_ref, c0_ref, g_ref):
    ef = ef_ref[...]
    h = (h0_ref[...]
         + lax.dot_general(ef, w1e_ref[...], _DN,
                           preferred_element_type=jnp.float32)
         + b1_ref[...])
    mu = jnp.mean(h, axis=1, keepdims=True)
    hc = h - mu
    var = jnp.mean(hc * hc, axis=1, keepdims=True)
    hn = hc * lax.rsqrt(var + 1e-5) * lng_ref[...] + lnb_ref[...]
    hr = jnp.maximum(hn, 0.0)
    gate = _sig(jnp.sum(ef * gw_ref[...], axis=1, keepdims=True) + gb_ref[...])
    g_ref[...] = (hr + c0_ref[...]) * gate


def _edge_mlp(h0, efp, w1e, b1, ln_g, ln_b, gate_w, gate_b, c0):
    e_pad, d = h0.shape
    de = efp.shape[1]
    be = 2048
    return pl.pallas_call(
        _edge_kernel,
        grid=(e_pad // be,),
        in_specs=[pl.BlockSpec((be, d), lambda i: (i, 0)),
                  pl.BlockSpec((be, de), lambda i: (i, 0)),
                  pl.BlockSpec((d, de), lambda i: (0, 0)),
                  pl.BlockSpec((1, d), lambda i: (0, 0)),
                  pl.BlockSpec((1, d), lambda i: (0, 0)),
                  pl.BlockSpec((1, d), lambda i: (0, 0)),
                  pl.BlockSpec((1, de), lambda i: (0, 0)),
                  pl.BlockSpec((1, 1), lambda i: (0, 0)),
                  pl.BlockSpec((1, d), lambda i: (0, 0))],
        out_specs=pl.BlockSpec((be, d), lambda i: (i, 0)),
        out_shape=jax.ShapeDtypeStruct((e_pad, d), jnp.float32),
    )(h0, efp, w1e, b1, ln_g, ln_b, gate_w, gate_b, c0)


def _update_kernel(nf_ref, gp_ref, w2_ref, wih_ref, whh_ref,
                   bih_ref, bhh_ref, out_ref):
    nfb = nf_ref[...]
    d = nfb.shape[1]
    g = gp_ref[0] + gp_ref[1]
    agg = lax.dot_general(g, w2_ref[...], _DN,
                          preferred_element_type=jnp.float32)
    gi = lax.dot_general(agg, wih_ref[...], _DN,
                         preferred_element_type=jnp.float32) + bih_ref[...]
    gh = lax.dot_general(nfb, whh_ref[...], _DN,
                         preferred_element_type=jnp.float32) + bhh_ref[...]
    r = _sig(gi[:, :d] + gh[:, :d])
    z = _sig(gi[:, d:2 * d] + gh[:, d:2 * d])
    n = jnp.tanh(gi[:, 2 * d:] + r * gh[:, 2 * d:])
    out_ref[...] = (1.0 - z) * n + z * nfb


def _gru_update(nf, gp, w2, wih, whh, bih, bhh):
    n, d = nf.shape
    nb = 2000 if n % 2000 == 0 else n
    return pl.pallas_call(
        _update_kernel,
        grid=(n // nb,),
        in_specs=[pl.BlockSpec((nb, d), lambda i: (i, 0)),
                  pl.BlockSpec((2, nb, d), lambda i: (0, i, 0)),
                  pl.BlockSpec((d, d), lambda i: (0, 0)),
                  pl.BlockSpec((3 * d, d), lambda i: (0, 0)),
                  pl.BlockSpec((3 * d, d), lambda i: (0, 0)),
                  pl.BlockSpec((1, 3 * d), lambda i: (0, 0)),
                  pl.BlockSpec((1, 3 * d), lambda i: (0, 0))],
        out_specs=pl.BlockSpec((nb, d), lambda i: (i, 0)),
        out_shape=jax.ShapeDtypeStruct((n, d), jnp.float32),
    )(nf, gp, w2, wih, whh, bih, bhh)


@functools.lru_cache(maxsize=None)
def _make_gather(n_nodes, d, e_pad):
    ch_per_w = e_pad // (_NW * _CHUNK)
    epw = ch_per_w * _CHUNK
    assert ch_per_w % 2 == 0 and ch_per_w >= 2
    mesh = plsc.VectorSubcoreMesh(core_axis_name="c", subcore_axis_name="s",
                                  num_cores=2, num_subcores=16)

    @functools.partial(
        pl.kernel,
        out_type=jax.ShapeDtypeStruct((e_pad, d), jnp.float32),
        mesh=mesh,
        scratch_types=[
            pltpu.VMEM((2, _CHUNK), jnp.int32),
            pltpu.VMEM((2, _CHUNK), jnp.int32),
            pltpu.VMEM((_CHUNK, d), jnp.float32),
            pltpu.VMEM((_CHUNK, d), jnp.float32),
            pltpu.VMEM((_CHUNK, d), jnp.float32),
            pltpu.VMEM((_CHUNK, d), jnp.float32),
            pltpu.SemaphoreType.DMA,
            pltpu.SemaphoreType.DMA,
        ],
    )
    def gather(ps_hbm, pd_hbm, src_hbm, dst_hbm, h0_hbm,
               sidx, didx, bufs_a, bufd_a, bufs_b, bufd_b, sem_a, sem_b):
        wid = lax.axis_index("s") * 2 + lax.axis_index("c")
        base = wid * epw
        nch = ch_per_w

        def load_idx(ci, b):
            e0 = base + ci * _CHUNK
            pltpu.sync_copy(src_hbm.at[pl.ds(e0, _CHUNK)], sidx.at[b])
            pltpu.sync_copy(dst_hbm.at[pl.ds(e0, _CHUNK)], didx.at[b])

        def fire(b, bs, bd, sem):
            pltpu.async_copy(ps_hbm.at[sidx.at[b]], bs, sem)
            pltpu.async_copy(pd_hbm.at[didx.at[b]], bd, sem)

        def drain(b, bs, bd, sem):
            pltpu.make_async_copy(ps_hbm.at[sidx.at[b]], bs, sem).wait()
            pltpu.make_async_copy(pd_hbm.at[didx.at[b]], bd, sem).wait()

        def add_and_store(ci, bs, bd):
            def row(i, c2):
                for j in range(d // _LANES):
                    sl = pl.ds(j * _LANES, _LANES)
                    bs[i, sl] = bs[i, sl] + bd[i, sl]
                return c2

            lax.fori_loop(0, _CHUNK, row, 0)
            pltpu.sync_copy(bs, h0_hbm.at[pl.ds(base + ci * _CHUNK, _CHUNK)])

        load_idx(0, 0)
        fire(0, bufs_a, bufd_a, sem_a)

        def pair(ci2, carry):
            ca = 2 * ci2
            load_idx(ca + 1, 1)
            fire(1, bufs_b, bufd_b, sem_b)
            drain(0, bufs_a, bufd_a, sem_a)
            add_and_store(ca, bufs_a, bufd_a)
            load_idx(jnp.minimum(ca + 2, nch - 2), 0)
            fire(0, bufs_a, bufd_a, sem_a)
            drain(1, bufs_b, bufd_b, sem_b)
            add_and_store(ca + 1, bufs_b, bufd_b)
            return carry

        lax.fori_loop(0, nch // 2, pair, 0)
        drain(0, bufs_a, bufd_a, sem_a)

    return gather


@functools.lru_cache(maxsize=None)
def _make_scatter(tbl, d, eh):
    ch_per_w = eh // (_NW * _CHUNK)
    epw = ch_per_w * _CHUNK
    rows_per_sub = tbl // 16
    zch = rows_per_sub // _CHUNK
    mesh = plsc.VectorSubcoreMesh(core_axis_name="c", subcore_axis_name="s",
                                  num_cores=2, num_subcores=16)

    assert ch_per_w % 2 == 0 and ch_per_w >= 2

    @functools.partial(
        pl.kernel,
        out_type=jax.ShapeDtypeStruct((2, tbl, d), jnp.float32),
        mesh=mesh,
        scratch_types=[
            pltpu.VMEM((2, _CHUNK), jnp.int32),
            pltpu.VMEM((_CHUNK, d), jnp.float32),
            pltpu.VMEM((_CHUNK, d), jnp.float32),
            pltpu.VMEM_SHARED((tbl, d), jnp.float32),
            pltpu.SemaphoreType.DMA,
            pltpu.SemaphoreType.DMA,
        ],
    )
    def scatter(ga_hbm, dsta_hbm, gb_hbm, dstb_hbm, gc_hbm, dstc_hbm,
                gd_hbm, dstd_hbm, outg_hbm,
                idx, buf_a, buf_b, gtab, sem_a, sem_b):
        cid = lax.axis_index("c")
        sid = lax.axis_index("s")
        wid = sid * 2 + cid
        base = wid * epw
        r0 = sid * rows_per_sub
        nch = ch_per_w

        def zrow(i, c2):
            for j in range(d // _LANES):
                buf_a[i, pl.ds(j * _LANES, _LANES)] = jnp.zeros((_LANES,),
                                                                jnp.float32)
            return c2

        lax.fori_loop(0, _CHUNK, zrow, 0)

        def zchunk(k, c2):
            pltpu.sync_copy(buf_a, gtab.at[pl.ds(r0 + k * _CHUNK, _CHUNK)])
            return c2

        lax.fori_loop(0, zch, zchunk, 0)
        plsc.subcore_barrier()

        def run_half(g_hbm, dst_hbm):
            def fire(ci, b, buf, sem):
                e0 = base + ci * _CHUNK
                pltpu.async_copy(dst_hbm.at[pl.ds(e0, _CHUNK)], idx.at[b], sem)
                pltpu.async_copy(g_hbm.at[pl.ds(e0, _CHUNK)], buf, sem)

            def drain(b, buf, sem):
                pltpu.make_async_copy(dst_hbm.at[pl.ds(base, _CHUNK)],
                                      idx.at[b], sem).wait()
                pltpu.make_async_copy(g_hbm.at[pl.ds(base, _CHUNK)],
                                      buf, sem).wait()

            fire(0, 0, buf_a, sem_a)

            def pair(ci2, carry):
                ca = 2 * ci2
                fire(ca + 1, 1, buf_b, sem_b)
                drain(0, buf_a, sem_a)
                pltpu.sync_copy(buf_a, gtab.at[idx.at[0]], add=True)
                fire(jnp.minimum(ca + 2, nch - 2), 0, buf_a, sem_a)
                drain(1, buf_b, sem_b)
                pltpu.sync_copy(buf_b, gtab.at[idx.at[1]], add=True)
                return carry

            lax.fori_loop(0, nch // 2, pair, 0)
            drain(0, buf_a, sem_a)

        run_half(ga_hbm, dsta_hbm)
        run_half(gb_hbm, dstb_hbm)
        run_half(gc_hbm, dstc_hbm)
        run_half(gd_hbm, dstd_hbm)
        plsc.subcore_barrier()

        def wchunk(k, c2):
            rr = r0 + k * _CHUNK
            pltpu.sync_copy(gtab.at[pl.ds(rr, _CHUNK)], buf_a)
            pltpu.sync_copy(buf_a, outg_hbm.at[cid, pl.ds(rr, _CHUNK)])
            return c2

        lax.fori_loop(0, zch, wchunk, 0)

    return scatter


def kernel(node_features, edge_indices, edge_features, W1, b1, ln_g, ln_b,
           W2, b2, gru_wih, gru_whh, gru_bih, gru_bhh, gate_w, gate_b):
    n, d = node_features.shape
    e, de = edge_features.shape

    chunks = -(-e // _CHUNK)
    chunks_pad = -(-chunks // (8 * _NW)) * (8 * _NW)
    e_pad = chunks_pad * _CHUNK
    eh = e_pad // 4
    pad = e_pad - e
    tbl = -(-(n + 1) // (16 * _CHUNK)) * (16 * _CHUNK)

    src = edge_indices[0]
    dst = edge_indices[1]
    gpad = jnp.arange(pad, dtype=jnp.int32) % n
    srcp = jnp.concatenate([src, gpad])
    dstg = jnp.concatenate([dst, gpad])
    pad_rows = n + jnp.arange(pad, dtype=jnp.int32) % (tbl - n)
    dsts = jnp.concatenate([dst, pad_rows])
    efp = jnp.concatenate([edge_features, jnp.zeros((pad, de), jnp.float32)],
                          axis=0)

    w1s = W1[:, :d]
    w1d = W1[:, d:2 * d]
    w1e = W1[:, 2 * d:]
    c0 = jnp.linalg.solve(W2, b2.astype(jnp.float32))

    ps, pd = _proj(node_features, w1s, w1d)

    gather = _make_gather(n, d, eh)
    mlp_args = (w1e, b1.reshape(1, d), ln_g.reshape(1, d),
                ln_b.reshape(1, d), gate_w.reshape(1, de),
                gate_b.reshape(1, 1), c0.reshape(1, d))
    gs = []
    scatter_args = []
    for q in range(4):
        lo, hi = q * eh, (q + 1) * eh
        h0q = gather(ps, pd, srcp[lo:hi], dstg[lo:hi])
        gq = _edge_mlp(h0q, efp[lo:hi], *mlp_args)
        gs.append(gq)
        scatter_args.extend([gq, dsts[lo:hi]])
    gp = _make_scatter(tbl, d, eh)(*scatter_args)
    out = _gru_update(node_features, gp, W2, gru_wih, gru_whh,
                      gru_bih.reshape(1, 3 * d), gru_bhh.reshape(1, 3 * d))
    return out


# speedup vs baseline: 1.0635x; 1.0635x over previous
import functools

import jax
import jax.numpy as jnp
from jax import lax
from jax.experimental import pallas as pl
from jax.experimental.pallas import tpu as pltpu
from jax.experimental.pallas import tpu_sc as plsc

_CHUNK = 128
_NW = 32
_LANES = 16


def _sig(x):
    return 1.0 / (1.0 + jnp.exp(-x))


_DN = (((1,), (1,)), ((), ()))


def _proj_kernel(nf_ref, w1s_ref, w1d_ref, ps_ref, pd_ref):
    nfb = nf_ref[...]
    ps_ref[...] = lax.dot_general(nfb, w1s_ref[...], _DN,
                                  preferred_element_type=jnp.float32)
    pd_ref[...] = lax.dot_general(nfb, w1d_ref[...], _DN,
                                  preferred_element_type=jnp.float32)


def _proj(nf, w1s, w1d):
    n, d = nf.shape
    nb = 2000 if n % 2000 == 0 else n
    return pl.pallas_call(
        _proj_kernel,
        grid=(n // nb,),
        in_specs=[pl.BlockSpec((nb, d), lambda i: (i, 0)),
                  pl.BlockSpec((d, d), lambda i: (0, 0)),
                  pl.BlockSpec((d, d), lambda i: (0, 0))],
        out_specs=[pl.BlockSpec((nb, d), lambda i: (i, 0)),
                   pl.BlockSpec((nb, d), lambda i: (i, 0))],
        out_shape=[jax.ShapeDtypeStruct((n, d), jnp.float32)] * 2,
    )(nf, w1s, w1d)


def _edge_kernel(h0_ref, ef_ref, w1e_ref, b1_ref, lng_ref, lnb_ref,
                 gw_ref, gb_ref, c0_ref, g_ref):
    ef = ef_ref[...]
    h = (h0_ref[...]
         + lax.dot_general(ef, w1e_ref[...], _DN,
                           preferred_element_type=jnp.float32)
         + b1_ref[...])
    mu = jnp.mean(h, axis=1, keepdims=True)
    hc = h - mu
    var = jnp.mean(hc * hc, axis=1, keepdims=True)
    hn = hc * lax.rsqrt(var + 1e-5) * lng_ref[...] + lnb_ref[...]
    hr = jnp.maximum(hn, 0.0)
    gate = _sig(jnp.sum(ef * gw_ref[...], axis=1, keepdims=True) + gb_ref[...])
    g_ref[...] = (hr + c0_ref[...]) * gate


def _edge_mlp(h0, efp, lo, w1e, b1, ln_g, ln_b, gate_w, gate_b, c0):
    eh, d = h0.shape
    de = efp.shape[1]
    be = 2048
    ob = lo // be
    return pl.pallas_call(
        _edge_kernel,
        grid=(eh // be,),
        in_specs=[pl.BlockSpec((be, d), lambda i: (i, 0)),
                  pl.BlockSpec((be, de), lambda i: (i + ob, 0)),
                  pl.BlockSpec((d, de), lambda i: (0, 0)),
                  pl.BlockSpec((1, d), lambda i: (0, 0)),
                  pl.BlockSpec((1, d), lambda i: (0, 0)),
                  pl.BlockSpec((1, d), lambda i: (0, 0)),
                  pl.BlockSpec((1, de), lambda i: (0, 0)),
                  pl.BlockSpec((1, 1), lambda i: (0, 0)),
                  pl.BlockSpec((1, d), lambda i: (0, 0))],
        out_specs=pl.BlockSpec((be, d), lambda i: (i, 0)),
        out_shape=jax.ShapeDtypeStruct((eh, d), jnp.float32),
    )(h0, efp, w1e, b1, ln_g, ln_b, gate_w, gate_b, c0)


def _update_kernel(nf_ref, gpa_ref, gpb_ref, w2_ref, wih_ref, whh_ref,
                   bih_ref, bhh_ref, out_ref):
    nfb = nf_ref[...]
    d = nfb.shape[1]
    g = (gpa_ref[0] + gpa_ref[1]) + (gpb_ref[0] + gpb_ref[1])
    agg = lax.dot_general(g, w2_ref[...], _DN,
                          preferred_element_type=jnp.float32)
    gi = lax.dot_general(agg, wih_ref[...], _DN,
                         preferred_element_type=jnp.float32) + bih_ref[...]
    gh = lax.dot_general(nfb, whh_ref[...], _DN,
                         preferred_element_type=jnp.float32) + bhh_ref[...]
    r = _sig(gi[:, :d] + gh[:, :d])
    z = _sig(gi[:, d:2 * d] + gh[:, d:2 * d])
    n = jnp.tanh(gi[:, 2 * d:] + r * gh[:, 2 * d:])
    out_ref[...] = (1.0 - z) * n + z * nfb


def _gru_update(nf, gpa, gpb, w2, wih, whh, bih, bhh):
    n, d = nf.shape
    nb = 2000 if n % 2000 == 0 else n
    return pl.pallas_call(
        _update_kernel,
        grid=(n // nb,),
        in_specs=[pl.BlockSpec((nb, d), lambda i: (i, 0)),
                  pl.BlockSpec((2, nb, d), lambda i: (0, i, 0)),
                  pl.BlockSpec((2, nb, d), lambda i: (0, i, 0)),
                  pl.BlockSpec((d, d), lambda i: (0, 0)),
                  pl.BlockSpec((3 * d, d), lambda i: (0, 0)),
                  pl.BlockSpec((3 * d, d), lambda i: (0, 0)),
                  pl.BlockSpec((1, 3 * d), lambda i: (0, 0)),
                  pl.BlockSpec((1, 3 * d), lambda i: (0, 0))],
        out_specs=pl.BlockSpec((nb, d), lambda i: (i, 0)),
        out_shape=jax.ShapeDtypeStruct((n, d), jnp.float32),
    )(nf, gpa, gpb, w2, wih, whh, bih, bhh)


@functools.lru_cache(maxsize=None)
def _make_gather(n_nodes, d, eh, part_off):
    ch_per_w = eh // (_NW * _CHUNK)
    epw = ch_per_w * _CHUNK
    assert ch_per_w % 2 == 0 and ch_per_w >= 2
    mesh = plsc.VectorSubcoreMesh(core_axis_name="c", subcore_axis_name="s",
                                  num_cores=2, num_subcores=16)

    @functools.partial(
        pl.kernel,
        out_type=jax.ShapeDtypeStruct((eh, d), jnp.float32),
        mesh=mesh,
        scratch_types=[
            pltpu.VMEM((2, _CHUNK), jnp.int32),
            pltpu.VMEM((2, _CHUNK), jnp.int32),
            pltpu.VMEM((_CHUNK, d), jnp.float32),
            pltpu.VMEM((_CHUNK, d), jnp.float32),
            pltpu.VMEM((_CHUNK, d), jnp.float32),
            pltpu.VMEM((_CHUNK, d), jnp.float32),
            pltpu.SemaphoreType.DMA,
            pltpu.SemaphoreType.DMA,
        ],
    )
    def gather(ps_hbm, pd_hbm, src_hbm, dst_hbm, h0_hbm,
               sidx, didx, bufs_a, bufd_a, bufs_b, bufd_b, sem_a, sem_b):
        wid = lax.axis_index("s") * 2 + lax.axis_index("c")
        base = wid * epw
        nch = ch_per_w

        def load_idx(ci, b):
            e0 = part_off + base + ci * _CHUNK
            pltpu.sync_copy(src_hbm.at[pl.ds(e0, _CHUNK)], sidx.at[b])
            pltpu.sync_copy(dst_hbm.at[pl.ds(e0, _CHUNK)], didx.at[b])

        def fire(b, bs, bd, sem):
            pltpu.async_copy(ps_hbm.at[sidx.at[b]], bs, sem)
            pltpu.async_copy(pd_hbm.at[didx.at[b]], bd, sem)

        def drain(b, bs, bd, sem):
            pltpu.make_async_copy(ps_hbm.at[sidx.at[b]], bs, sem).wait()
            pltpu.make_async_copy(pd_hbm.at[didx.at[b]], bd, sem).wait()

        def add_and_store(ci, bs, bd):
            def row(i, c2):
                for j in range(d // _LANES):
                    sl = pl.ds(j * _LANES, _LANES)
                    bs[i, sl] = bs[i, sl] + bd[i, sl]
                return c2

            lax.fori_loop(0, _CHUNK, row, 0)
            pltpu.sync_copy(bs, h0_hbm.at[pl.ds(base + ci * _CHUNK, _CHUNK)])

        load_idx(0, 0)
        fire(0, bufs_a, bufd_a, sem_a)

        def pair(ci2, carry):
            ca = 2 * ci2
            load_idx(ca + 1, 1)
            fire(1, bufs_b, bufd_b, sem_b)
            drain(0, bufs_a, bufd_a, sem_a)
            add_and_store(ca, bufs_a, bufd_a)
            load_idx(jnp.minimum(ca + 2, nch - 2), 0)
            fire(0, bufs_a, bufd_a, sem_a)
            drain(1, bufs_b, bufd_b, sem_b)
            add_and_store(ca + 1, bufs_b, bufd_b)
            return carry

        lax.fori_loop(0, nch // 2, pair, 0)
        drain(0, bufs_a, bufd_a, sem_a)

    return gather


@functools.lru_cache(maxsize=None)
def _make_scatter(tbl, d, eh, part_off):
    ch_per_w = eh // (_NW * _CHUNK)
    epw = ch_per_w * _CHUNK
    rows_per_sub = tbl // 16
    zch = rows_per_sub // _CHUNK
    mesh = plsc.VectorSubcoreMesh(core_axis_name="c", subcore_axis_name="s",
                                  num_cores=2, num_subcores=16)

    assert ch_per_w % 2 == 0 and ch_per_w >= 2

    @functools.partial(
        pl.kernel,
        out_type=jax.ShapeDtypeStruct((2, tbl, d), jnp.float32),
        mesh=mesh,
        scratch_types=[
            pltpu.VMEM((2, _CHUNK), jnp.int32),
            pltpu.VMEM((_CHUNK, d), jnp.float32),
            pltpu.VMEM((_CHUNK, d), jnp.float32),
            pltpu.VMEM_SHARED((tbl, d), jnp.float32),
            pltpu.SemaphoreType.DMA,
            pltpu.SemaphoreType.DMA,
        ],
    )
    def scatter(g_hbm, dst_hbm, outg_hbm,
                idx, buf_a, buf_b, gtab, sem_a, sem_b):
        cid = lax.axis_index("c")
        sid = lax.axis_index("s")
        wid = sid * 2 + cid
        base = wid * epw
        r0 = sid * rows_per_sub
        nch = ch_per_w

        def zrow(i, c2):
            for j in range(d // _LANES):
                buf_a[i, pl.ds(j * _LANES, _LANES)] = jnp.zeros((_LANES,),
                                                                jnp.float32)
            return c2

        lax.fori_loop(0, _CHUNK, zrow, 0)

        def zchunk(k, c2):
            pltpu.sync_copy(buf_a, gtab.at[pl.ds(r0 + k * _CHUNK, _CHUNK)])
            return c2

        lax.fori_loop(0, zch, zchunk, 0)
        plsc.subcore_barrier()

        def fire(ci, b, buf, sem):
            e0 = base + ci * _CHUNK
            pltpu.async_copy(dst_hbm.at[pl.ds(part_off + e0, _CHUNK)],
                             idx.at[b], sem)
            pltpu.async_copy(g_hbm.at[pl.ds(e0, _CHUNK)], buf, sem)

        def drain(b, buf, sem):
            pltpu.make_async_copy(dst_hbm.at[pl.ds(part_off + base, _CHUNK)],
                                  idx.at[b], sem).wait()
            pltpu.make_async_copy(g_hbm.at[pl.ds(base, _CHUNK)],
                                  buf, sem).wait()

        fire(0, 0, buf_a, sem_a)

        def pair(ci2, carry):
            ca = 2 * ci2
            fire(ca + 1, 1, buf_b, sem_b)
            drain(0, buf_a, sem_a)
            pltpu.sync_copy(buf_a, gtab.at[idx.at[0]], add=True)
            fire(jnp.minimum(ca + 2, nch - 2), 0, buf_a, sem_a)
            drain(1, buf_b, sem_b)
            pltpu.sync_copy(buf_b, gtab.at[idx.at[1]], add=True)
            return carry

        lax.fori_loop(0, nch // 2, pair, 0)
        drain(0, buf_a, sem_a)
        plsc.subcore_barrier()

        def wchunk(k, c2):
            rr = r0 + k * _CHUNK
            pltpu.sync_copy(gtab.at[pl.ds(rr, _CHUNK)], buf_a)
            pltpu.sync_copy(buf_a, outg_hbm.at[cid, pl.ds(rr, _CHUNK)])
            return c2

        lax.fori_loop(0, zch, wchunk, 0)

    return scatter


def kernel(node_features, edge_indices, edge_features, W1, b1, ln_g, ln_b,
           W2, b2, gru_wih, gru_whh, gru_bih, gru_bhh, gate_w, gate_b):
    n, d = node_features.shape
    e, de = edge_features.shape

    chunks = -(-e // _CHUNK)
    chunks_pad = -(-chunks // (4 * _NW)) * (4 * _NW)
    e_pad = chunks_pad * _CHUNK
    eh = e_pad // 2
    pad = e_pad - e
    tbl = -(-(n + 1) // (16 * _CHUNK)) * (16 * _CHUNK)

    src = edge_indices[0]
    dst = edge_indices[1]
    gpad = jnp.arange(pad, dtype=jnp.int32) % n
    srcp = jnp.concatenate([src, gpad])
    dstg = jnp.concatenate([dst, gpad])
    pad_rows = n + jnp.arange(pad, dtype=jnp.int32) % (tbl - n)
    dsts = jnp.concatenate([dst, pad_rows])
    efp = jnp.concatenate([edge_features, jnp.zeros((pad, de), jnp.float32)],
                          axis=0)

    w1s = W1[:, :d]
    w1d = W1[:, d:2 * d]
    w1e = W1[:, 2 * d:]
    c0 = jnp.linalg.solve(W2, b2.astype(jnp.float32))

    ps, pd = _proj(node_features, w1s, w1d)

    mlp_args = (w1e, b1.reshape(1, d), ln_g.reshape(1, d),
                ln_b.reshape(1, d), gate_w.reshape(1, de),
                gate_b.reshape(1, 1), c0.reshape(1, d))
    h0a = _make_gather(n, d, eh, 0)(ps, pd, srcp, dstg)
    h0b = _make_gather(n, d, eh, eh)(ps, pd, srcp, dstg)
    ga = _edge_mlp(h0a, efp, 0, *mlp_args)
    gb = _edge_mlp(h0b, efp, eh, *mlp_args)
    gpa = _make_scatter(tbl, d, eh, 0)(ga, dsts)
    gpb = _make_scatter(tbl, d, eh, eh)(gb, dsts)
    out = _gru_update(node_features, gpa, gpb, W2, gru_wih, gru_whh,
                      gru_bih.reshape(1, 3 * d), gru_bhh.reshape(1, 3 * d))
    return out
